# Initial kernel scaffold; baseline (speedup 1.0000x reference)
import jax
import jax.numpy as jnp
from jax.experimental import pallas as pl


def kernel(output, target, anchors):
    raise NotImplementedError("write your pallas kernel here")


# speedup vs baseline: 10.2182x; 10.2182x over previous
import jax
import jax.numpy as jnp
from jax import lax
from jax.experimental import pallas as pl
from jax.experimental.pallas import tpu as pltpu

_NUM_CLASSES = 80
_STRIDE = 32
_A = 5
_THRESH = 0.6
_OBJECT_SCALE = 5.0
_NOOBJECT_SCALE = 1.0


def _body(x_ref, t_ref, a_ref, o_ref):
    f32 = jnp.float32
    H = W = 32
    N = t_ref.shape[1]
    r = lax.broadcasted_iota(jnp.int32, (8, 128), 0)
    c = lax.broadcasted_iota(jnp.int32, (8, 128), 1)
    pos = r * 128 + c
    gx = (pos % W).astype(f32)
    gy = (pos // W).astype(f32)

    acc = jnp.zeros((8, 128), f32)
    for a in range(_A):
        tx = x_ref[0, a, 0]
        ty = x_ref[0, a, 1]
        tw = x_ref[0, a, 2]
        th = x_ref[0, a, 3]
        conf = x_ref[0, a, 4]
        aw = a_ref[a, 0]
        ah = a_ref[a, 1]

        px = (1.0 / (1.0 + jnp.exp(-tx)) + gx) * float(_STRIDE)
        py = (1.0 / (1.0 + jnp.exp(-ty)) + gy) * float(_STRIDE)
        pw = jnp.exp(tw) * aw
        ph = jnp.exp(th) * ah
        p1x = px - pw / 2
        p1y = py - ph / 2
        p2x = px + pw / 2
        p2y = py + ph / 2
        parea = (p2x - p1x) * (p2y - p1y)

        def step(j, carry):
            best_iou, bx, by, bw, bh, bcls = carry
            gcls = t_ref[0, j, 0]
            gcx = t_ref[0, j, 1]
            gcy = t_ref[0, j, 2]
            gw = t_ref[0, j, 3]
            gh = t_ref[0, j, 4]
            g1x = gcx - gw / 2
            g1y = gcy - gh / 2
            g2x = gcx + gw / 2
            g2y = gcy + gh / 2
            garea = (g2x - g1x) * (g2y - g1y)
            x1 = jnp.maximum(g1x, p1x)
            y1 = jnp.maximum(g1y, p1y)
            x2 = jnp.minimum(g2x, p2x)
            y2 = jnp.minimum(g2y, p2y)
            inter = jnp.maximum(x2 - x1, 0.0) * jnp.maximum(y2 - y1, 0.0)
            union = garea + parea - inter + 1e-6
            iou = inter / union
            upd = iou > best_iou
            return (
                jnp.maximum(iou, best_iou),
                jnp.where(upd, gcx, bx),
                jnp.where(upd, gcy, by),
                jnp.where(upd, gw, bw),
                jnp.where(upd, gh, bh),
                jnp.where(upd, gcls, bcls),
            )

        z = jnp.zeros((8, 128), f32)
        best_iou, bx, by, bw, bh, bcls = lax.fori_loop(
            0, N, step, (jnp.full((8, 128), -1.0, f32), z, z, z, z, z)
        )

        mask = best_iou > _THRESH
        cm = jnp.where(mask, 1.0, 0.0)
        scale = jnp.where(mask, _OBJECT_SCALE, _NOOBJECT_SCALE)

        dx = tx * cm - bx * cm
        dy = ty * cm - by * cm
        dw = tw * cm - bw * cm
        dh = th * cm - bh * cm
        coord_l = dx * dx + dy * dy + dw * dw + dh * dh

        dc = conf * scale - cm * scale
        conf_l = dc * dc

        m = x_ref[0, a, 5]
        for ci in range(1, _NUM_CLASSES):
            m = jnp.maximum(m, x_ref[0, a, 5 + ci])
        ssum = jnp.zeros((8, 128), f32)
        picked = jnp.zeros((8, 128), f32)
        for ci in range(_NUM_CLASSES):
            v = x_ref[0, a, 5 + ci]
            ssum = ssum + jnp.exp(v - m)
            picked = jnp.where(bcls == float(ci), v, picked)
        ce = jnp.log(ssum) - (picked - m)
        cls_l = cm * ce

        acc = acc + coord_l + conf_l + cls_l
    o_ref[0, 0, 0] = jnp.sum(acc)


def kernel(output, target, anchors):
    B = output.shape[0]
    x = output.reshape(B, _A, 5 + _NUM_CLASSES, 8, 128)
    partial = pl.pallas_call(
        _body,
        grid=(B,),
        in_specs=[
            pl.BlockSpec((1, _A, 5 + _NUM_CLASSES, 8, 128), lambda b: (b, 0, 0, 0, 0)),
            pl.BlockSpec((1, target.shape[1], 5), lambda b: (b, 0, 0), memory_space=pltpu.SMEM),
            pl.BlockSpec((_A, 2), lambda b: (0, 0), memory_space=pltpu.SMEM),
        ],
        out_specs=pl.BlockSpec((1, 1, 1), lambda b: (b, 0, 0), memory_space=pltpu.SMEM),
        out_shape=jax.ShapeDtypeStruct((B, 1, 1), jnp.float32),
    )(x, target, anchors)
    return jnp.sum(partial)


# speedup vs baseline: 16.4497x; 1.6098x over previous
import jax
import jax.numpy as jnp
from jax import lax
from jax.experimental import pallas as pl
from jax.experimental.pallas import tpu as pltpu

_NUM_CLASSES = 80
_STRIDE = 32
_A = 5
_THRESH = 0.6
_OBJECT_SCALE = 5.0
_NOOBJECT_SCALE = 1.0


def _body(x_ref, t_ref, a_ref, o_ref):
    f32 = jnp.float32
    H = W = 32
    N = t_ref.shape[1]
    r = lax.broadcasted_iota(jnp.int32, (8, 128), 0)
    c = lax.broadcasted_iota(jnp.int32, (8, 128), 1)
    pos = r * 128 + c
    gx = (pos % W).astype(f32)
    gy = (pos // W).astype(f32)

    gt = []
    for j in range(N):
        gcls = t_ref[0, j, 0]
        gcx = t_ref[0, j, 1]
        gcy = t_ref[0, j, 2]
        gw = t_ref[0, j, 3]
        gh = t_ref[0, j, 4]
        g1x = gcx - gw / 2
        g1y = gcy - gh / 2
        g2x = gcx + gw / 2
        g2y = gcy + gh / 2
        garea = (g2x - g1x) * (g2y - g1y)
        gt.append((gcls, gcx, gcy, gw, gh, g1x, g1y, g2x, g2y, garea))

    acc = jnp.zeros((8, 128), f32)
    for a in range(_A):
        tx = x_ref[0, a, 0]
        ty = x_ref[0, a, 1]
        tw = x_ref[0, a, 2]
        th = x_ref[0, a, 3]
        conf = x_ref[0, a, 4]
        aw = a_ref[a, 0]
        ah = a_ref[a, 1]

        px = (1.0 / (1.0 + jnp.exp(-tx)) + gx) * float(_STRIDE)
        py = (1.0 / (1.0 + jnp.exp(-ty)) + gy) * float(_STRIDE)
        pw = jnp.exp(tw) * aw
        ph = jnp.exp(th) * ah
        p1x = px - pw / 2
        p1y = py - ph / 2
        p2x = px + pw / 2
        p2y = py + ph / 2
        parea = (p2x - p1x) * (p2y - p1y)

        z = jnp.zeros((8, 128), f32)
        best_iou = jnp.full((8, 128), -1.0, f32)
        bx, by, bw, bh, bcls = z, z, z, z, z
        for j in range(N):
            gcls, gcx, gcy, gw, gh, g1x, g1y, g2x, g2y, garea = gt[j]
            x1 = jnp.maximum(g1x, p1x)
            y1 = jnp.maximum(g1y, p1y)
            x2 = jnp.minimum(g2x, p2x)
            y2 = jnp.minimum(g2y, p2y)
            inter = jnp.maximum(x2 - x1, 0.0) * jnp.maximum(y2 - y1, 0.0)
            union = garea + parea - inter + 1e-6
            iou = inter / union
            upd = iou > best_iou
            best_iou = jnp.maximum(iou, best_iou)
            bx = jnp.where(upd, gcx, bx)
            by = jnp.where(upd, gcy, by)
            bw = jnp.where(upd, gw, bw)
            bh = jnp.where(upd, gh, bh)
            bcls = jnp.where(upd, gcls, bcls)

        mask = best_iou > _THRESH
        cm = jnp.where(mask, 1.0, 0.0)
        scale = jnp.where(mask, _OBJECT_SCALE, _NOOBJECT_SCALE)

        dx = tx * cm - bx * cm
        dy = ty * cm - by * cm
        dw = tw * cm - bw * cm
        dh = th * cm - bh * cm
        coord_l = dx * dx + dy * dy + dw * dw + dh * dh

        dc = conf * scale - cm * scale
        conf_l = dc * dc

        m = x_ref[0, a, 5]
        for ci in range(1, _NUM_CLASSES):
            m = jnp.maximum(m, x_ref[0, a, 5 + ci])
        ssum = jnp.zeros((8, 128), f32)
        picked = jnp.zeros((8, 128), f32)
        for ci in range(_NUM_CLASSES):
            v = x_ref[0, a, 5 + ci]
            ssum = ssum + jnp.exp(v - m)
            picked = jnp.where(bcls == float(ci), v, picked)
        ce = jnp.log(ssum) - (picked - m)
        cls_l = cm * ce

        acc = acc + coord_l + conf_l + cls_l
    o_ref[0, 0, 0] = jnp.sum(acc)


def kernel(output, target, anchors):
    B = output.shape[0]
    x = output.reshape(B, _A, 5 + _NUM_CLASSES, 8, 128)
    partial = pl.pallas_call(
        _body,
        grid=(B,),
        in_specs=[
            pl.BlockSpec((1, _A, 5 + _NUM_CLASSES, 8, 128), lambda b: (b, 0, 0, 0, 0)),
            pl.BlockSpec((1, target.shape[1], 5), lambda b: (b, 0, 0), memory_space=pltpu.SMEM),
            pl.BlockSpec((_A, 2), lambda b: (0, 0), memory_space=pltpu.SMEM),
        ],
        out_specs=pl.BlockSpec((1, 1, 1), lambda b: (b, 0, 0), memory_space=pltpu.SMEM),
        out_shape=jax.ShapeDtypeStruct((B, 1, 1), jnp.float32),
    )(x, target, anchors)
    return jnp.sum(partial)
